# Initial kernel scaffold; baseline (speedup 1.0000x reference)
import functools

import jax
import jax.numpy as jnp
from jax import lax
from jax.experimental import pallas as pl
from jax.experimental.pallas import tpu as pltpu
from jax.experimental.pallas import tpu_sc as plsc

_NC = 2
_NS = 16
_NW = _NC * _NS
_L = 16
_HP = 64
_CH = 80

_MESH = plsc.VectorSubcoreMesh(core_axis_name="c", subcore_axis_name="s")


@functools.lru_cache(maxsize=None)
def _deg_kernel(n, e_per_w, ch):
    nchunk = e_per_w // ch
    rpt = n // _NS
    w = _L

    def body(dst_hbm, out_hbm, dstv, onesv, zbuf, acc):
        c = lax.axis_index("c")
        s = lax.axis_index("s")
        wid = s * _NC + c
        one = jnp.ones((_L,), jnp.float32)
        zero = jnp.zeros((_L,), jnp.float32)

        def initrow(i, _):
            onesv[i] = one
            return 0
        lax.fori_loop(0, ch, initrow, 0)

        def zrow(i, _):
            zbuf[i] = zero
            return 0
        lax.fori_loop(0, rpt, zrow, 0)

        pltpu.sync_copy(zbuf, acc.at[pl.ds(s * rpt, rpt)])
        plsc.subcore_barrier()
        pltpu.sync_copy(dst_hbm.at[wid], dstv)

        def chunk(i, _):
            pltpu.sync_copy(onesv, acc.at[dstv.at[i]], add=True)
            return 0
        lax.fori_loop(0, nchunk, chunk, 0)

        plsc.subcore_barrier()
        pltpu.sync_copy(acc.at[pl.ds(s * rpt, rpt)],
                        out_hbm.at[c, pl.ds(s * rpt, rpt)])

    return pl.kernel(
        body,
        out_type=jax.ShapeDtypeStruct((_NC, n, w), jnp.float32),
        mesh=_MESH,
        scratch_types=[
            pltpu.VMEM((nchunk, ch), jnp.int32),
            pltpu.VMEM((ch, w), jnp.float32),
            pltpu.VMEM((rpt, w), jnp.float32),
            pltpu.VMEM_SHARED((n, w), jnp.float32),
        ],
    )


@functools.lru_cache(maxsize=None)
def _scatter_kernel(n, e_per_w, ch, hp):
    nchunk = e_per_w // ch
    rpt = n // _NS
    ng = hp // _L

    def body(y_hbm, src_hbm, dst_hbm, out_hbm, srcv, dstv, rows, zbuf, acc, sem):
        c = lax.axis_index("c")
        s = lax.axis_index("s")
        wid = s * _NC + c
        zero = jnp.zeros((_L,), jnp.float32)

        def zrow(i, _):
            for g in range(ng):
                zbuf[i, pl.ds(g * _L, _L)] = zero
            return 0
        lax.fori_loop(0, rpt, zrow, 0)

        pltpu.sync_copy(zbuf, acc.at[pl.ds(s * rpt, rpt)])
        plsc.subcore_barrier()
        pltpu.sync_copy(src_hbm.at[wid], srcv)
        pltpu.sync_copy(dst_hbm.at[wid], dstv)

        def chunk(i, _):
            pltpu.async_copy(y_hbm.at[srcv.at[i]], rows, sem).wait()
            pltpu.sync_copy(rows, acc.at[dstv.at[i]], add=True)
            return 0
        lax.fori_loop(0, nchunk, chunk, 0)

        plsc.subcore_barrier()
        pltpu.sync_copy(acc.at[pl.ds(s * rpt, rpt)],
                        out_hbm.at[c, pl.ds(s * rpt, rpt)])

    return pl.kernel(
        body,
        out_type=jax.ShapeDtypeStruct((_NC, n, hp), jnp.float32),
        mesh=_MESH,
        scratch_types=[
            pltpu.VMEM((nchunk, ch), jnp.int32),
            pltpu.VMEM((nchunk, ch), jnp.int32),
            pltpu.VMEM((ch, hp), jnp.float32),
            pltpu.VMEM((rpt, hp), jnp.float32),
            pltpu.VMEM_SHARED((n, hp), jnp.float32),
            pltpu.SemaphoreType.DMA,
        ],
    )


@functools.lru_cache(maxsize=None)
def _edge_kernel(n, e_per_w, ch, hp, h):
    nchunk = e_per_w // ch
    ng_e = ch // _L

    def body(a_hbm, b_hbm, src_hbm, dst_hbm, w_hbm, bias_hbm, out_hbm,
             srcv, dstv, arows, brows, outv, wv, biasv, sema, semb):
        c = lax.axis_index("c")
        s = lax.axis_index("s")
        wid = s * _NC + c
        pltpu.sync_copy(src_hbm.at[wid], srcv)
        pltpu.sync_copy(dst_hbm.at[wid], dstv)
        pltpu.sync_copy(w_hbm, wv)
        pltpu.sync_copy(bias_hbm, biasv)
        bias = biasv[...]
        iota = lax.iota(jnp.int32, _L)

        def chunk(i, _):
            ca = pltpu.async_copy(a_hbm.at[srcv.at[i]], arows, sema)
            cb = pltpu.async_copy(b_hbm.at[dstv.at[i]], brows, semb)
            ca.wait()
            cb.wait()
            accs = [jnp.zeros((_L,), jnp.float32) for _ in range(ng_e)]
            for j in range(h):
                colj = jnp.full((_L,), j, jnp.int32)
                wj = plsc.load_gather(wv, [colj])
                for g in range(ng_e):
                    rid = iota + (g * _L)
                    av = plsc.load_gather(arows, [rid, colj])
                    bv = plsc.load_gather(brows, [rid, colj])
                    accs[g] = accs[g] + jnp.maximum(av + bv, 0.0) * wj
            for g in range(ng_e):
                outv[i, pl.ds(g * _L, _L)] = accs[g] + bias
            return 0
        lax.fori_loop(0, nchunk, chunk, 0)

        pltpu.sync_copy(outv, out_hbm.at[wid])

    return pl.kernel(
        body,
        out_type=jax.ShapeDtypeStruct((_NW, nchunk, ch), jnp.float32),
        mesh=_MESH,
        scratch_types=[
            pltpu.VMEM((nchunk, ch), jnp.int32),
            pltpu.VMEM((nchunk, ch), jnp.int32),
            pltpu.VMEM((ch, hp), jnp.float32),
            pltpu.VMEM((ch, hp), jnp.float32),
            pltpu.VMEM((nchunk, ch), jnp.float32),
            pltpu.VMEM((hp,), jnp.float32),
            pltpu.VMEM((_L,), jnp.float32),
            pltpu.SemaphoreType.DMA,
            pltpu.SemaphoreType.DMA,
        ],
    )


def _tc_first(x, w1p, degp):
    def body(x_ref, w_ref, d_ref, o_ref):
        dd = d_ref[...]
        deg = dd[0, :, 0] + dd[1, :, 0] + 1.0
        dinv = lax.rsqrt(deg)
        o_ref[...] = jnp.dot(x_ref[...], w_ref[...],
                             preferred_element_type=jnp.float32) * dinv[:, None]
    return pl.pallas_call(
        body,
        out_shape=jax.ShapeDtypeStruct((x.shape[0], w1p.shape[1]), jnp.float32),
    )(x, w1p, degp)


def _tc_mid(p, y, degp, bp, wp):
    def body(p_ref, y_ref, d_ref, b_ref, w_ref, o_ref):
        dd = d_ref[...]
        deg = dd[0, :, 0] + dd[1, :, 0] + 1.0
        dinv = lax.rsqrt(deg)[:, None]
        pp = p_ref[...]
        pre = (pp[0] + pp[1] + y_ref[...]) * dinv + b_ref[...]
        hmat = jnp.maximum(pre, 0.0)
        o_ref[...] = jnp.dot(hmat, w_ref[...],
                             preferred_element_type=jnp.float32) * dinv
    return pl.pallas_call(
        body,
        out_shape=jax.ShapeDtypeStruct((y.shape[0], wp.shape[1]), jnp.float32),
    )(p, y, degp, bp, wp)


def _tc_head(p, y, degp, bp, wap, wbp, l1bp):
    def body(p_ref, y_ref, d_ref, b_ref, wa_ref, wb_ref, lb_ref, a_ref, bb_ref):
        dd = d_ref[...]
        deg = dd[0, :, 0] + dd[1, :, 0] + 1.0
        dinv = lax.rsqrt(deg)[:, None]
        pp = p_ref[...]
        pre = (pp[0] + pp[1] + y_ref[...]) * dinv + b_ref[...]
        hmat = jnp.maximum(pre, 0.0)
        a_ref[...] = jnp.dot(hmat, wa_ref[...],
                             preferred_element_type=jnp.float32) + lb_ref[...]
        bb_ref[...] = jnp.dot(hmat, wb_ref[...],
                              preferred_element_type=jnp.float32)
    nshape = jax.ShapeDtypeStruct((y.shape[0], wap.shape[1]), jnp.float32)
    return pl.pallas_call(
        body,
        out_shape=(nshape, nshape),
    )(p, y, degp, bp, wap, wbp, l1bp)


def kernel(x, g_edge_index, lg_edge_index, W1, b1, W2, b2, L1W, L1b, L2W, L2b):
    n, d = x.shape
    e = g_edge_index.shape[1]
    h = W1.shape[1]
    hp = _HP
    ch = _CH
    e_per_w = e // _NW
    nchunk = e_per_w // ch
    assert e_per_w * _NW == e and nchunk * ch == e_per_w and n % _NS == 0

    src = g_edge_index[0].astype(jnp.int32).reshape(_NW, nchunk, ch)
    dst = g_edge_index[1].astype(jnp.int32).reshape(_NW, nchunk, ch)

    f32 = jnp.float32
    w1p = jnp.zeros((d, hp), f32).at[:, :h].set(W1)
    w2p = jnp.zeros((hp, hp), f32).at[:h, :h].set(W2)
    wap = jnp.zeros((hp, hp), f32).at[:h, :h].set(L1W[:h])
    wbp = jnp.zeros((hp, hp), f32).at[:h, :h].set(L1W[h:])
    b1p = jnp.zeros((1, hp), f32).at[0, :h].set(b1)
    b2p = jnp.zeros((1, hp), f32).at[0, :h].set(b2)
    l1bp = jnp.zeros((1, hp), f32).at[0, :h].set(L1b)
    wvec = jnp.zeros((hp,), f32).at[:h].set(L2W[:, 0])
    bias16 = jnp.full((_L,), L2b[0], f32)

    degp = _deg_kernel(n, e_per_w, ch)(dst)
    y1 = _tc_first(x, w1p, degp)
    p1 = _scatter_kernel(n, e_per_w, ch, hp)(y1, src, dst)
    y2 = _tc_mid(p1, y1, degp, b1p, w2p)
    p2 = _scatter_kernel(n, e_per_w, ch, hp)(y2, src, dst)
    a_t, b_t = _tc_head(p2, y2, degp, b2p, wap, wbp, l1bp)
    oute = _edge_kernel(n, e_per_w, ch, hp, h)(a_t, b_t, src, dst, wvec, bias16)
    return oute.reshape(e, 1)


# speedup vs baseline: 9.9705x; 9.9705x over previous
import functools

import jax
import jax.numpy as jnp
from jax import lax
from jax.experimental import pallas as pl
from jax.experimental.pallas import tpu as pltpu
from jax.experimental.pallas import tpu_sc as plsc

_NC = 2
_NS = 16
_NW = _NC * _NS
_L = 16
_HP = 64
_CH = 80

_MESH = plsc.VectorSubcoreMesh(core_axis_name="c", subcore_axis_name="s")
_SC_PARAMS = pltpu.CompilerParams(use_tc_tiling_on_sc=False, needs_layout_passes=False)


@functools.lru_cache(maxsize=None)
def _deg_kernel(npad, e_per_w, ch):
    nchunk = e_per_w // ch
    rpt = npad // _NS
    w = _L

    def body(dst_hbm, out_hbm, dstv, onesv, zbuf, acc):
        c = lax.axis_index("c")
        s = lax.axis_index("s")
        wid = s * _NC + c
        one = jnp.ones((_L,), jnp.float32)
        zero = jnp.zeros((_L,), jnp.float32)

        def initrow(i, _):
            onesv[i] = one
            return 0
        lax.fori_loop(0, ch, initrow, 0)

        def zrow(i, _):
            zbuf[i] = zero
            return 0
        lax.fori_loop(0, rpt, zrow, 0)

        pltpu.sync_copy(zbuf, acc.at[pl.ds(s * rpt, rpt)])
        plsc.subcore_barrier()
        pltpu.sync_copy(dst_hbm.at[wid], dstv)

        def chunk(i, _):
            pltpu.sync_copy(onesv, acc.at[dstv.at[i]], add=True)
            return 0
        lax.fori_loop(0, nchunk, chunk, 0)

        plsc.subcore_barrier()
        pltpu.sync_copy(acc.at[pl.ds(s * rpt, rpt)],
                        out_hbm.at[c, pl.ds(s * rpt, rpt)])

    return pl.kernel(
        body,
        out_type=jax.ShapeDtypeStruct((_NC, npad, w), jnp.float32),
        mesh=_MESH,
        compiler_params=_SC_PARAMS,
        scratch_types=[
            pltpu.VMEM((nchunk, ch), jnp.int32),
            pltpu.VMEM((ch, w), jnp.float32),
            pltpu.VMEM((rpt, w), jnp.float32),
            pltpu.VMEM_SHARED((npad, w), jnp.float32),
        ],
    )


@functools.lru_cache(maxsize=None)
def _scatter_kernel(npad, e_per_w, ch, hp):
    nchunk = e_per_w // ch
    rpt = npad // _NS
    ng = hp // _L

    def body(y_hbm, src_hbm, dst_hbm, out_hbm, srcv, dstv, rows, zbuf, acc, sem):
        c = lax.axis_index("c")
        s = lax.axis_index("s")
        wid = s * _NC + c
        zero = jnp.zeros((_L,), jnp.float32)

        def zrow(i, _):
            for g in range(ng):
                zbuf[i, pl.ds(g * _L, _L)] = zero
            return 0
        lax.fori_loop(0, rpt, zrow, 0)

        pltpu.sync_copy(zbuf, acc.at[pl.ds(s * rpt, rpt)])
        plsc.subcore_barrier()
        pltpu.sync_copy(src_hbm.at[wid], srcv)
        pltpu.sync_copy(dst_hbm.at[wid], dstv)

        def chunk(i, _):
            pltpu.async_copy(y_hbm.at[srcv.at[i]], rows, sem).wait()
            pltpu.sync_copy(rows, acc.at[dstv.at[i]], add=True)
            return 0
        lax.fori_loop(0, nchunk, chunk, 0)

        plsc.subcore_barrier()
        pltpu.sync_copy(acc.at[pl.ds(s * rpt, rpt)],
                        out_hbm.at[c, pl.ds(s * rpt, rpt)])

    return pl.kernel(
        body,
        out_type=jax.ShapeDtypeStruct((_NC, npad, hp), jnp.float32),
        mesh=_MESH,
        compiler_params=_SC_PARAMS,
        scratch_types=[
            pltpu.VMEM((nchunk, ch), jnp.int32),
            pltpu.VMEM((nchunk, ch), jnp.int32),
            pltpu.VMEM((ch, hp), jnp.float32),
            pltpu.VMEM((rpt, hp), jnp.float32),
            pltpu.VMEM_SHARED((npad, hp), jnp.float32),
            pltpu.SemaphoreType.DMA,
        ],
    )


@functools.lru_cache(maxsize=None)
def _edge_kernel(e_per_w, ch, hp, h):
    nchunk = e_per_w // ch
    ng_e = ch // _L

    def body(a_hbm, b_hbm, src_hbm, dst_hbm, w_hbm, bias_hbm, out_hbm,
             srcv, dstv, arows, brows, outv, wv, biasv, sema, semb):
        c = lax.axis_index("c")
        s = lax.axis_index("s")
        wid = s * _NC + c
        pltpu.sync_copy(src_hbm.at[wid], srcv)
        pltpu.sync_copy(dst_hbm.at[wid], dstv)
        pltpu.sync_copy(w_hbm, wv)
        pltpu.sync_copy(bias_hbm, biasv)
        bias = biasv[...]
        iota = lax.iota(jnp.int32, _L)

        def chunk(i, _):
            ca = pltpu.async_copy(a_hbm.at[srcv.at[i]], arows, sema)
            cb = pltpu.async_copy(b_hbm.at[dstv.at[i]], brows, semb)
            ca.wait()
            cb.wait()
            accs = [jnp.zeros((_L,), jnp.float32) for _ in range(ng_e)]
            for j in range(h):
                colj = jnp.full((_L,), j, jnp.int32)
                wj = plsc.load_gather(wv, [jnp.full((_L,), j + 1, jnp.int32)])
                for g in range(ng_e):
                    rid = iota + (g * _L)
                    av = plsc.load_gather(arows, [rid, colj])
                    bv = plsc.load_gather(brows, [rid, colj])
                    accs[g] = accs[g] + jnp.maximum(av + bv, 0.0) * wj
            for g in range(ng_e):
                outv[i, pl.ds(g * _L, _L)] = accs[g] + bias
            return 0
        lax.fori_loop(0, nchunk, chunk, 0)

        pltpu.sync_copy(outv, out_hbm.at[wid])

    return pl.kernel(
        body,
        out_type=jax.ShapeDtypeStruct((_NW, nchunk, ch), jnp.float32),
        mesh=_MESH,
        compiler_params=_SC_PARAMS,
        scratch_types=[
            pltpu.VMEM((nchunk, ch), jnp.int32),
            pltpu.VMEM((nchunk, ch), jnp.int32),
            pltpu.VMEM((ch, hp), jnp.float32),
            pltpu.VMEM((ch, hp), jnp.float32),
            pltpu.VMEM((nchunk, ch), jnp.float32),
            pltpu.VMEM((hp,), jnp.float32),
            pltpu.VMEM((_L,), jnp.float32),
            pltpu.SemaphoreType.DMA,
            pltpu.SemaphoreType.DMA,
        ],
    )


def _tc_first(x, w1p, degp):
    n = x.shape[0]

    def body(x_ref, w_ref, d_ref, o_ref):
        dd = d_ref[...]
        deg = dd[0, :n, 0] + dd[1, :n, 0] + 1.0
        dinv = 1.0 / jnp.sqrt(deg)
        o_ref[...] = jnp.dot(x_ref[...], w_ref[...],
                             preferred_element_type=jnp.float32) * dinv[:, None]
    return pl.pallas_call(
        body,
        out_shape=jax.ShapeDtypeStruct((x.shape[0], w1p.shape[1]), jnp.float32),
    )(x, w1p, degp)


def _tc_mid(p, y, degp, bp, wp):
    n = y.shape[0]

    def body(p_ref, y_ref, d_ref, b_ref, w_ref, o_ref):
        dd = d_ref[...]
        deg = dd[0, :n, 0] + dd[1, :n, 0] + 1.0
        dinv = (1.0 / jnp.sqrt(deg))[:, None]
        pp = p_ref[...]
        pre = (pp[0, :n] + pp[1, :n] + y_ref[...]) * dinv + b_ref[...]
        hmat = jnp.maximum(pre, 0.0)
        o_ref[...] = jnp.dot(hmat, w_ref[...],
                             preferred_element_type=jnp.float32) * dinv
    return pl.pallas_call(
        body,
        out_shape=jax.ShapeDtypeStruct((y.shape[0], wp.shape[1]), jnp.float32),
    )(p, y, degp, bp, wp)


def _tc_head(p, y, degp, bp, wap, wbp, l1bp):
    n = y.shape[0]

    def body(p_ref, y_ref, d_ref, b_ref, wa_ref, wb_ref, lb_ref, a_ref, bb_ref):
        dd = d_ref[...]
        deg = dd[0, :n, 0] + dd[1, :n, 0] + 1.0
        dinv = (1.0 / jnp.sqrt(deg))[:, None]
        pp = p_ref[...]
        pre = (pp[0, :n] + pp[1, :n] + y_ref[...]) * dinv + b_ref[...]
        hmat = jnp.maximum(pre, 0.0)
        a_ref[...] = jnp.dot(hmat, wa_ref[...],
                             preferred_element_type=jnp.float32) + lb_ref[...]
        bb_ref[...] = jnp.dot(hmat, wb_ref[...],
                              preferred_element_type=jnp.float32)
    nshape = jax.ShapeDtypeStruct((y.shape[0], wap.shape[1]), jnp.float32)
    return pl.pallas_call(
        body,
        out_shape=(nshape, nshape),
    )(p, y, degp, bp, wap, wbp, l1bp)


def kernel(x, g_edge_index, lg_edge_index, W1, b1, W2, b2, L1W, L1b, L2W, L2b):
    n, d = x.shape
    e = g_edge_index.shape[1]
    h = W1.shape[1]
    hp = _HP
    ch = _CH
    e_per_w = e // _NW
    nchunk = e_per_w // ch
    npad = ((n + 127) // 128) * 128
    assert e_per_w * _NW == e and nchunk * ch == e_per_w

    src = g_edge_index[0].astype(jnp.int32).reshape(_NW, nchunk, ch)
    dst = g_edge_index[1].astype(jnp.int32).reshape(_NW, nchunk, ch)

    f32 = jnp.float32
    w1p = jnp.zeros((d, hp), f32).at[:, :h].set(W1)
    w2p = jnp.zeros((hp, hp), f32).at[:h, :h].set(W2)
    wap = jnp.zeros((hp, hp), f32).at[:h, :h].set(L1W[:h])
    wbp = jnp.zeros((hp, hp), f32).at[:h, :h].set(L1W[h:])
    b1p = jnp.zeros((1, hp), f32).at[0, :h].set(b1)
    b2p = jnp.zeros((1, hp), f32).at[0, :h].set(b2)
    l1bp = jnp.zeros((1, hp), f32).at[0, :h].set(L1b)
    wvec = jnp.zeros((hp,), f32).at[1:h + 1].set(L2W[:, 0])
    bias16 = jnp.full((_L,), L2b[0], f32)

    degp = _deg_kernel(npad, e_per_w, ch)(dst)
    y1 = _tc_first(x, w1p, degp)
    p1 = _scatter_kernel(npad, e_per_w, ch, hp)(y1, src, dst)
    y2 = _tc_mid(p1, y1, degp, b1p, w2p)
    p2 = _scatter_kernel(npad, e_per_w, ch, hp)(y2, src, dst)
    a_t, b_t = _tc_head(p2, y2, degp, b2p, wap, wbp, l1bp)
    oute = _edge_kernel(e_per_w, ch, hp, h)(a_t, b_t, src, dst, wvec, bias16)
    return oute.reshape(e, 1)
